# Initial kernel scaffold; baseline (speedup 1.0000x reference)
import jax
import jax.numpy as jnp
from jax.experimental import pallas as pl


def kernel(seqs, species, vocab_table, cat_table, cat_scale):
    raise NotImplementedError("write your pallas kernel here")


# speedup vs baseline: 7.3688x; 7.3688x over previous
import functools

import jax
import jax.numpy as jnp
from jax import lax
from jax.experimental import pallas as pl
from jax.experimental.pallas import tpu as pltpu

try:
    from jax.experimental.pallas import tpu_sc as plsc
except ImportError:
    plsc = None


def _species_gather(cat_table, species):
    b = species.shape[0]
    d = cat_table.shape[1]
    nw = 32
    bpw = b // nw

    mesh = plsc.VectorSubcoreMesh(core_axis_name="c", subcore_axis_name="s")

    @functools.partial(
        pl.kernel,
        mesh=mesh,
        compiler_params=pltpu.CompilerParams(use_tc_tiling_on_sc=False),
        out_type=jax.ShapeDtypeStruct((b, d), jnp.float32),
        scratch_types=[
            pltpu.VMEM((bpw,), jnp.int32),
            pltpu.VMEM((bpw, d), jnp.float32),
            pltpu.SemaphoreType.DMA,
        ],
    )
    def gather_kernel(table_hbm, idx_hbm, out_hbm, idx_v, rows_v, sem):
        wid = lax.axis_index("s") * 2 + lax.axis_index("c")
        base = wid * bpw
        pltpu.sync_copy(idx_hbm.at[pl.ds(base, bpw)], idx_v)
        pltpu.async_copy(table_hbm.at[idx_v], rows_v, sem).wait()
        pltpu.sync_copy(rows_v, out_hbm.at[pl.ds(base, bpw)])

    return gather_kernel(cat_table, species)


def _combine_body(scale_ref, seqs_ref, vt_ref, se_ref, out_ref):
    bb, l, e = out_ref.shape
    s = seqs_ref[...]
    vt = vt_ref[...]
    se = se_ref[...]
    scale = scale_ref[0]

    s3 = lax.broadcast_in_dim(s, (bb, l, e), (0, 1))
    se3 = lax.broadcast_in_dim(se, (bb, l, e), (0, 2))
    r = lax.broadcast_in_dim(vt[0], (bb, l, e), (2,))
    for v in range(1, 5):
        row = lax.broadcast_in_dim(vt[v], (bb, l, e), (2,))
        r = jnp.where(s3 == v, row, r)
    out_ref[...] = r + scale * se3


def _combine(seqs, vocab_table, species_emb, cat_scale, bb=64):
    b, l = seqs.shape
    e = vocab_table.shape[1]
    vt_pad = jnp.zeros((8, e), jnp.float32).at[: vocab_table.shape[0]].set(
        vocab_table
    )
    scale_arr = jnp.reshape(cat_scale.astype(jnp.float32), (1,))

    grid = (b // bb,)
    return pl.pallas_call(
        _combine_body,
        grid=grid,
        in_specs=[
            pl.BlockSpec(memory_space=pltpu.SMEM),
            pl.BlockSpec((bb, l), lambda i: (i, 0)),
            pl.BlockSpec((8, e), lambda i: (0, 0)),
            pl.BlockSpec((bb, e), lambda i: (i, 0)),
        ],
        out_specs=pl.BlockSpec((bb, l, e), lambda i: (i, 0, 0)),
        out_shape=jax.ShapeDtypeStruct((b, l, e), jnp.float32),
    )(scale_arr, seqs, vt_pad, species_emb)


@jax.jit
def kernel(seqs, species, vocab_table, cat_table, cat_scale):
    seqs = seqs.astype(jnp.int32)
    species = species.astype(jnp.int32)
    species_emb = _species_gather(cat_table, species)
    total_emb = _combine(seqs, vocab_table, species_emb, cat_scale)
    return (total_emb, species_emb)


# speedup vs baseline: 30.3576x; 4.1198x over previous
import functools

import jax
import jax.numpy as jnp
from jax import lax
from jax.experimental import pallas as pl
from jax.experimental.pallas import tpu as pltpu

try:
    from jax.experimental.pallas import tpu_sc as plsc
except ImportError:
    plsc = None


def _species_gather(cat_table, species):
    b = species.shape[0]
    d = cat_table.shape[1]
    nw = 32
    bpw = b // nw

    mesh = plsc.VectorSubcoreMesh(core_axis_name="c", subcore_axis_name="s")

    @functools.partial(
        pl.kernel,
        mesh=mesh,
        compiler_params=pltpu.CompilerParams(use_tc_tiling_on_sc=False),
        out_type=jax.ShapeDtypeStruct((b, d), jnp.float32),
        scratch_types=[
            pltpu.VMEM((bpw,), jnp.int32),
            pltpu.VMEM((bpw, d), jnp.float32),
            pltpu.SemaphoreType.DMA,
        ],
    )
    def gather_kernel(table_hbm, idx_hbm, out_hbm, idx_v, rows_v, sem):
        wid = lax.axis_index("s") * 2 + lax.axis_index("c")
        base = wid * bpw
        pltpu.sync_copy(idx_hbm.at[pl.ds(base, bpw)], idx_v)
        pltpu.async_copy(table_hbm.at[idx_v], rows_v, sem).wait()
        pltpu.sync_copy(rows_v, out_hbm.at[pl.ds(base, bpw)])

    return gather_kernel(cat_table, species)


def _combine_body(scale_ref, seqs_ref, vt_ref, se_ref, out_ref):
    ll, e, bbl = out_ref.shape
    s = seqs_ref[...]
    vt = vt_ref[...]
    sef = se_ref[...] * scale_ref[0]

    vplanes = [jnp.broadcast_to(vt[:, v : v + 1], (e, bbl)) for v in range(5)]
    for j in range(ll):
        s2 = jnp.broadcast_to(s[j : j + 1, :], (e, bbl))
        r = vplanes[0]
        for v in range(1, 5):
            r = jnp.where(s2 == v, vplanes[v], r)
        out_ref[j] = r + sef


def _combine(seqs, vocab_table, species_emb, cat_scale, ll=8, bbl=2048):
    b, l = seqs.shape
    e = vocab_table.shape[1]
    vt_pad = jnp.zeros((e, 8), jnp.float32).at[:, :5].set(vocab_table.T)
    seqs_t = jnp.transpose(seqs)
    se_t = jnp.transpose(species_emb)
    scale_arr = jnp.reshape(cat_scale.astype(jnp.float32), (1,))

    grid = (b // bbl, l // ll)
    out_t = pl.pallas_call(
        _combine_body,
        grid=grid,
        in_specs=[
            pl.BlockSpec(memory_space=pltpu.SMEM),
            pl.BlockSpec((ll, bbl), lambda ib, il: (il, ib)),
            pl.BlockSpec((e, 8), lambda ib, il: (0, 0)),
            pl.BlockSpec((e, bbl), lambda ib, il: (0, ib)),
        ],
        out_specs=pl.BlockSpec((ll, e, bbl), lambda ib, il: (il, 0, ib)),
        out_shape=jax.ShapeDtypeStruct((l, e, b), jnp.float32),
    )(scale_arr, seqs_t, vt_pad, se_t)
    return jnp.transpose(out_t, (2, 0, 1))


@jax.jit
def kernel(seqs, species, vocab_table, cat_table, cat_scale):
    seqs = seqs.astype(jnp.int32)
    species = species.astype(jnp.int32)
    species_emb = _species_gather(cat_table, species)
    total_emb = _combine(seqs, vocab_table, species_emb, cat_scale)
    return (total_emb, species_emb)


# speedup vs baseline: 30.8702x; 1.0169x over previous
import functools

import jax
import jax.numpy as jnp
from jax import lax
from jax.experimental import pallas as pl
from jax.experimental.pallas import tpu as pltpu

try:
    from jax.experimental.pallas import tpu_sc as plsc
except ImportError:
    plsc = None


def _species_gather(cat_table, species):
    b = species.shape[0]
    d = cat_table.shape[1]
    nw = 32
    bpw = b // nw

    mesh = plsc.VectorSubcoreMesh(core_axis_name="c", subcore_axis_name="s")

    @functools.partial(
        pl.kernel,
        mesh=mesh,
        compiler_params=pltpu.CompilerParams(use_tc_tiling_on_sc=False),
        out_type=jax.ShapeDtypeStruct((b, d), jnp.float32),
        scratch_types=[
            pltpu.VMEM((bpw,), jnp.int32),
            pltpu.VMEM((bpw, d), jnp.float32),
            pltpu.SemaphoreType.DMA,
        ],
    )
    def gather_kernel(table_hbm, idx_hbm, out_hbm, idx_v, rows_v, sem):
        wid = lax.axis_index("s") * 2 + lax.axis_index("c")
        base = wid * bpw
        pltpu.sync_copy(idx_hbm.at[pl.ds(base, bpw)], idx_v)
        pltpu.async_copy(table_hbm.at[idx_v], rows_v, sem).wait()
        pltpu.sync_copy(rows_v, out_hbm.at[pl.ds(base, bpw)])

    return gather_kernel(cat_table, species)


def _combine_body(scale_ref, seqs_ref, vt_ref, se_ref, out_ref):
    ll, e, bbl = out_ref.shape
    s = seqs_ref[...]
    vt = vt_ref[...]
    sef = se_ref[...] * scale_ref[0]

    vplanes = [jnp.broadcast_to(vt[:, v : v + 1], (e, bbl)) for v in range(5)]
    for j in range(ll):
        s2 = jnp.broadcast_to(s[j], (e, bbl))
        r = vplanes[0]
        for v in range(1, 5):
            r = jnp.where(s2 == v, vplanes[v], r)
        out_ref[j] = r + sef


def _combine(seqs, vocab_table, species_emb, cat_scale, ll=8, bbl=4096):
    b, l = seqs.shape
    e = vocab_table.shape[1]
    vt_pad = jnp.zeros((e, 8), jnp.float32).at[:, :5].set(vocab_table.T)
    seqs_t = jnp.reshape(jnp.transpose(seqs), (l, 1, b))
    se_t = jnp.transpose(species_emb)
    scale_arr = jnp.reshape(cat_scale.astype(jnp.float32), (1,))

    grid = (b // bbl, l // ll)
    out_t = pl.pallas_call(
        _combine_body,
        grid=grid,
        in_specs=[
            pl.BlockSpec(memory_space=pltpu.SMEM),
            pl.BlockSpec((ll, 1, bbl), lambda ib, il: (il, 0, ib)),
            pl.BlockSpec((e, 8), lambda ib, il: (0, 0)),
            pl.BlockSpec((e, bbl), lambda ib, il: (0, ib)),
        ],
        out_specs=pl.BlockSpec((ll, e, bbl), lambda ib, il: (il, 0, ib)),
        out_shape=jax.ShapeDtypeStruct((l, e, b), jnp.float32),
    )(scale_arr, seqs_t, vt_pad, se_t)
    return jnp.transpose(out_t, (2, 0, 1))


@jax.jit
def kernel(seqs, species, vocab_table, cat_table, cat_scale):
    seqs = seqs.astype(jnp.int32)
    species = species.astype(jnp.int32)
    species_emb = _species_gather(cat_table, species)
    total_emb = _combine(seqs, vocab_table, species_emb, cat_scale)
    return (total_emb, species_emb)
